# Initial kernel scaffold; baseline (speedup 1.0000x reference)
import functools

import jax
import jax.numpy as jnp
from jax import lax
from jax.experimental import pallas as pl
from jax.experimental.pallas import tpu as pltpu
from jax.experimental.pallas import tpu_sc as plsc

_NC = 2
_NS = 16
_NW = _NC * _NS

_IW = 125


def _sc_gather_pair(table, idx_a2d, idx_b2d):
    n_rows = idx_a2d.shape[0]
    E = n_rows * _IW
    per_w = E // _NW
    CH = 8 * _IW
    iters = per_w // CH
    mesh = plsc.VectorSubcoreMesh(core_axis_name="c", subcore_axis_name="s")

    @functools.partial(
        pl.kernel,
        out_type=(
            jax.ShapeDtypeStruct((E, 4), jnp.float32),
            jax.ShapeDtypeStruct((E, 4), jnp.float32),
        ),
        mesh=mesh,
        scratch_types=[
            pltpu.VMEM((8, _IW), jnp.int32),
            pltpu.VMEM((8, _IW), jnp.int32),
            pltpu.VMEM((CH, 4), jnp.float32),
            pltpu.VMEM((CH, 4), jnp.float32),
            pltpu.SemaphoreType.DMA,
            pltpu.SemaphoreType.DMA,
        ],
    )
    def k(x_hbm, ia_hbm, ib_hbm, oa_hbm, ob_hbm, ia_v, ib_v, ra_v, rb_v,
          sem_a, sem_b):
        c = lax.axis_index("c")
        s = lax.axis_index("s")
        wid = s * _NC + c
        base = wid * per_w

        def body(it, carry):
            eoff = base + it * CH
            roff = eoff // _IW
            pltpu.sync_copy(ia_hbm.at[pl.ds(roff, 8), :], ia_v)
            pltpu.sync_copy(ib_hbm.at[pl.ds(roff, 8), :], ib_v)
            cps = []
            for j in range(8):
                cps.append(pltpu.async_copy(
                    x_hbm.at[ia_v.at[j]],
                    ra_v.at[pl.ds(j * _IW, _IW), :], sem_a))
                cps.append(pltpu.async_copy(
                    x_hbm.at[ib_v.at[j]],
                    rb_v.at[pl.ds(j * _IW, _IW), :], sem_b))
            for cp in cps:
                cp.wait()
            pltpu.sync_copy(ra_v, oa_hbm.at[pl.ds(eoff, CH), :])
            pltpu.sync_copy(rb_v, ob_hbm.at[pl.ds(eoff, CH), :])
            return carry

        lax.fori_loop(0, iters, body, 0)

    return k(table, idx_a2d, idx_b2d)


def _sc_segment_sum(dst2d, msg, zeros_n4):
    N = zeros_n4.shape[0]
    n_rows = dst2d.shape[0]
    E = n_rows * _IW
    per_sc = E // _NC
    per_tile = per_sc // _NS
    CH = 16 * _IW
    iters = per_tile // CH
    per_tile_n = N // _NS
    mesh = plsc.VectorSubcoreMesh(core_axis_name="c", subcore_axis_name="s")

    @functools.partial(
        pl.kernel,
        out_type=jax.ShapeDtypeStruct((_NC, N, 4), jnp.float32),
        mesh=mesh,
        scratch_types=[
            pltpu.VMEM((16, _IW), jnp.int32),
            pltpu.VMEM((CH, 4), jnp.float32),
            pltpu.VMEM_SHARED((N, 4), jnp.float32),
        ],
    )
    def k(d_hbm, m_hbm, z_hbm, out_hbm, idx_v, upd_v, acc_sh):
        c = lax.axis_index("c")
        s = lax.axis_index("s")
        noff = s * per_tile_n
        pltpu.sync_copy(z_hbm.at[pl.ds(noff, per_tile_n), :],
                        acc_sh.at[pl.ds(noff, per_tile_n), :])
        plsc.subcore_barrier()
        base = c * per_sc + s * per_tile

        def body(it, carry):
            eoff = base + it * CH
            roff = eoff // _IW
            pltpu.sync_copy(d_hbm.at[pl.ds(roff, 16), :], idx_v)
            pltpu.sync_copy(m_hbm.at[pl.ds(eoff, CH), :], upd_v)
            for j in range(16):
                pltpu.sync_copy(upd_v.at[pl.ds(j * _IW, _IW), :],
                                acc_sh.at[idx_v.at[j]], add=True)
            return carry

        lax.fori_loop(0, iters, body, 0)
        plsc.subcore_barrier()
        pltpu.sync_copy(acc_sh.at[pl.ds(noff, per_tile_n), :],
                        out_hbm.at[c, pl.ds(noff, per_tile_n), :])

    return k(dst2d, msg, zeros_n4)


def _edge_mlp_body(gi, gj, ea, w1, b1, w2, b2, w3, b3, out, *, sigmoid):
    m = jnp.concatenate([gi[...], gj[...], ea[...]], axis=1)
    h = jnp.dot(m, w1[...], preferred_element_type=jnp.float32) + b1[...]
    h = jnp.maximum(h, 0.0)
    h = jnp.dot(h, w2[...], preferred_element_type=jnp.float32) + b2[...]
    h = jnp.maximum(h, 0.0)
    o = jnp.dot(h, w3[...], preferred_element_type=jnp.float32) + b3[...]
    if sigmoid:
        o = jax.nn.sigmoid(o)
    out[...] = o


def _edge_mlp(gi, gj, ea, w1, b1, w2, b2, w3, b3, *, sigmoid, bm=2048):
    E = gi.shape[0]
    n_out = w3.shape[1]
    eb = lambda n: pl.BlockSpec((bm, n), lambda i: (i, 0))
    wb = lambda a: pl.BlockSpec(a.shape, lambda i: (0, 0))
    return pl.pallas_call(
        functools.partial(_edge_mlp_body, sigmoid=sigmoid),
        grid=(E // bm,),
        in_specs=[eb(4), eb(4), eb(4),
                  wb(w1), wb(b1), wb(w2), wb(b2), wb(w3), wb(b3)],
        out_specs=eb(n_out),
        out_shape=jax.ShapeDtypeStruct((E, n_out), jnp.float32),
    )(gi, gj, ea, w1, b1, w2, b2, w3, b3)


def _node_mlp_body(x, p0, p1, w1, b1, w2, b2, w3, b3, out):
    m = jnp.concatenate([x[...], p0[...] + p1[...]], axis=1)
    h = jnp.dot(m, w1[...], preferred_element_type=jnp.float32) + b1[...]
    h = jnp.maximum(h, 0.0)
    h = jnp.dot(h, w2[...], preferred_element_type=jnp.float32) + b2[...]
    h = jnp.maximum(h, 0.0)
    out[...] = jnp.dot(h, w3[...], preferred_element_type=jnp.float32) + b3[...]


def _node_mlp(x, p0, p1, w1, b1, w2, b2, w3, b3, *, bn=2000):
    N = x.shape[0]
    eb = lambda n: pl.BlockSpec((bn, n), lambda i: (i, 0))
    wb = lambda a: pl.BlockSpec(a.shape, lambda i: (0, 0))
    return pl.pallas_call(
        _node_mlp_body,
        grid=(N // bn,),
        in_specs=[eb(3), eb(4), eb(4),
                  wb(w1), wb(b1), wb(w2), wb(b2), wb(w3), wb(b3)],
        out_specs=eb(4),
        out_shape=jax.ShapeDtypeStruct((N, 4), jnp.float32),
    )(x, p0, p1, w1, b1, w2, b2, w3, b3)


def _cat_w1(W1):
    z = jnp.zeros((1, W1.shape[1]), W1.dtype)
    return jnp.concatenate([W1[0:3], z, W1[3:6], z, W1[6:10]], axis=0)


def kernel(x, edge_index, edge_attr,
           R1_W1, R1_b1, R1_W2, R1_b2, R1_W3, R1_b3,
           O_W1, O_b1, O_W2, O_b2, O_W3, O_b3,
           R2_W1, R2_b1, R2_W2, R2_b2, R2_W3, R2_b3):
    N = x.shape[0]
    E = edge_index.shape[1]

    xpad = jnp.concatenate([x, jnp.zeros((N, 1), x.dtype)], axis=1)
    dst2d = edge_index[1].reshape(E // _IW, _IW)
    src2d = edge_index[0].reshape(E // _IW, _IW)
    zeros_n4 = jnp.zeros((N, 4), jnp.float32)

    gi, gj = _sc_gather_pair(xpad, dst2d, src2d)
    emsg = _edge_mlp(
        gi, gj, edge_attr,
        _cat_w1(R1_W1), R1_b1.reshape(1, -1),
        R1_W2, R1_b2.reshape(1, -1),
        R1_W3, R1_b3.reshape(1, -1),
        sigmoid=False)

    parts = _sc_segment_sum(dst2d, emsg, zeros_n4)
    w3p = jnp.concatenate([O_W3, jnp.zeros((O_W3.shape[0], 1), O_W3.dtype)],
                          axis=1)
    b3p = jnp.concatenate([O_b3, jnp.zeros((1,), O_b3.dtype)])
    xt_pad = _node_mlp(
        x, parts[0], parts[1],
        O_W1, O_b1.reshape(1, -1),
        O_W2, O_b2.reshape(1, -1),
        w3p, b3p.reshape(1, -1))

    gi2, gj2 = _sc_gather_pair(xt_pad, dst2d, src2d)
    return _edge_mlp(
        gi2, gj2, emsg,
        _cat_w1(R2_W1), R2_b1.reshape(1, -1),
        R2_W2, R2_b2.reshape(1, -1),
        R2_W3, R2_b3.reshape(1, -1),
        sigmoid=True)


# speedup vs baseline: 4.0619x; 4.0619x over previous
import functools

import jax
import jax.numpy as jnp
from jax import lax
from jax.experimental import pallas as pl
from jax.experimental.pallas import tpu as pltpu
from jax.experimental.pallas import tpu_sc as plsc

_NC = 2
_NS = 16
_NW = _NC * _NS

_IW = 125


def _sc_gather_pair(table, idx_a2d, idx_b2d):
    n_rows = idx_a2d.shape[0]
    E = n_rows * _IW
    per_w = E // _NW
    CH = 8 * _IW
    iters = per_w // CH
    mesh = plsc.VectorSubcoreMesh(core_axis_name="c", subcore_axis_name="s")

    @functools.partial(
        pl.kernel,
        out_type=(
            jax.ShapeDtypeStruct((E, 4), jnp.float32),
            jax.ShapeDtypeStruct((E, 4), jnp.float32),
        ),
        mesh=mesh,
        scratch_types=[
            pltpu.VMEM((8, _IW), jnp.int32),
            pltpu.VMEM((8, _IW), jnp.int32),
            pltpu.VMEM((CH, 4), jnp.float32),
            pltpu.VMEM((CH, 4), jnp.float32),
            pltpu.SemaphoreType.DMA,
            pltpu.SemaphoreType.DMA,
        ],
        compiler_params=pltpu.CompilerParams(use_tc_tiling_on_sc=False),
    )
    def k(x_hbm, ia_hbm, ib_hbm, oa_hbm, ob_hbm, ia_v, ib_v, ra_v, rb_v,
          sem_a, sem_b):
        c = lax.axis_index("c")
        s = lax.axis_index("s")
        wid = s * _NC + c
        base = wid * per_w

        def body(it, carry):
            eoff = pl.multiple_of(base + it * CH, CH)
            roff = pl.multiple_of(eoff // _IW, 8)
            pltpu.sync_copy(ia_hbm.at[pl.ds(roff, 8), :], ia_v)
            pltpu.sync_copy(ib_hbm.at[pl.ds(roff, 8), :], ib_v)
            cps = []
            for j in range(8):
                cps.append(pltpu.async_copy(
                    x_hbm.at[ia_v.at[j]],
                    ra_v.at[pl.ds(j * _IW, _IW), :], sem_a))
                cps.append(pltpu.async_copy(
                    x_hbm.at[ib_v.at[j]],
                    rb_v.at[pl.ds(j * _IW, _IW), :], sem_b))
            for cp in cps:
                cp.wait()
            pltpu.sync_copy(ra_v, oa_hbm.at[pl.ds(eoff, CH), :])
            pltpu.sync_copy(rb_v, ob_hbm.at[pl.ds(eoff, CH), :])
            return carry

        lax.fori_loop(0, iters, body, 0)

    return k(table, idx_a2d, idx_b2d)


def _sc_segment_sum(dst2d, msg, zeros_n4):
    N = zeros_n4.shape[0]
    n_rows = dst2d.shape[0]
    E = n_rows * _IW
    per_sc = E // _NC
    per_tile = per_sc // _NS
    CH = 16 * _IW
    iters = per_tile // CH
    per_tile_n = N // _NS
    mesh = plsc.VectorSubcoreMesh(core_axis_name="c", subcore_axis_name="s")

    @functools.partial(
        pl.kernel,
        out_type=jax.ShapeDtypeStruct((_NC, N, 4), jnp.float32),
        mesh=mesh,
        scratch_types=[
            pltpu.VMEM((16, _IW), jnp.int32),
            pltpu.VMEM((CH, 4), jnp.float32),
            pltpu.VMEM_SHARED((N, 4), jnp.float32),
        ],
        compiler_params=pltpu.CompilerParams(use_tc_tiling_on_sc=False),
    )
    def k(d_hbm, m_hbm, z_hbm, out_hbm, idx_v, upd_v, acc_sh):
        c = lax.axis_index("c")
        s = lax.axis_index("s")
        noff = s * per_tile_n
        pltpu.sync_copy(z_hbm.at[pl.ds(noff, per_tile_n), :],
                        acc_sh.at[pl.ds(noff, per_tile_n), :])
        plsc.subcore_barrier()
        base = c * per_sc + s * per_tile

        def body(it, carry):
            eoff = pl.multiple_of(base + it * CH, CH)
            roff = pl.multiple_of(eoff // _IW, 16)
            pltpu.sync_copy(d_hbm.at[pl.ds(roff, 16), :], idx_v)
            pltpu.sync_copy(m_hbm.at[pl.ds(eoff, CH), :], upd_v)
            for j in range(16):
                pltpu.sync_copy(upd_v.at[pl.ds(j * _IW, _IW), :],
                                acc_sh.at[idx_v.at[j]], add=True)
            return carry

        lax.fori_loop(0, iters, body, 0)
        plsc.subcore_barrier()
        pltpu.sync_copy(acc_sh.at[pl.ds(noff, per_tile_n), :],
                        out_hbm.at[c, pl.ds(noff, per_tile_n), :])

    return k(dst2d, msg, zeros_n4)


def _edge_mlp_body(gi, gj, ea, w1, b1, w2, b2, w3, b3, out, *, sigmoid):
    m = jnp.concatenate([gi[...], gj[...], ea[...]], axis=1)
    h = jnp.dot(m, w1[...], preferred_element_type=jnp.float32) + b1[...]
    h = jnp.maximum(h, 0.0)
    h = jnp.dot(h, w2[...], preferred_element_type=jnp.float32) + b2[...]
    h = jnp.maximum(h, 0.0)
    o = jnp.dot(h, w3[...], preferred_element_type=jnp.float32) + b3[...]
    if sigmoid:
        o = jax.nn.sigmoid(o)
    out[...] = o


def _edge_mlp(gi, gj, ea, w1, b1, w2, b2, w3, b3, *, sigmoid, bm=2048):
    E = gi.shape[0]
    n_out = w3.shape[1]
    eb = lambda n: pl.BlockSpec((bm, n), lambda i: (i, 0))
    wb = lambda a: pl.BlockSpec(a.shape, lambda i: (0, 0))
    return pl.pallas_call(
        functools.partial(_edge_mlp_body, sigmoid=sigmoid),
        grid=(E // bm,),
        in_specs=[eb(4), eb(4), eb(4),
                  wb(w1), wb(b1), wb(w2), wb(b2), wb(w3), wb(b3)],
        out_specs=eb(n_out),
        out_shape=jax.ShapeDtypeStruct((E, n_out), jnp.float32),
    )(gi, gj, ea, w1, b1, w2, b2, w3, b3)


def _node_mlp_body(x, p0, p1, w1, b1, w2, b2, w3, b3, out):
    m = jnp.concatenate([x[...], p0[...] + p1[...]], axis=1)
    h = jnp.dot(m, w1[...], preferred_element_type=jnp.float32) + b1[...]
    h = jnp.maximum(h, 0.0)
    h = jnp.dot(h, w2[...], preferred_element_type=jnp.float32) + b2[...]
    h = jnp.maximum(h, 0.0)
    out[...] = jnp.dot(h, w3[...], preferred_element_type=jnp.float32) + b3[...]


def _node_mlp(x, p0, p1, w1, b1, w2, b2, w3, b3, *, bn=2000):
    N = x.shape[0]
    eb = lambda n: pl.BlockSpec((bn, n), lambda i: (i, 0))
    wb = lambda a: pl.BlockSpec(a.shape, lambda i: (0, 0))
    return pl.pallas_call(
        _node_mlp_body,
        grid=(N // bn,),
        in_specs=[eb(3), eb(4), eb(4),
                  wb(w1), wb(b1), wb(w2), wb(b2), wb(w3), wb(b3)],
        out_specs=eb(4),
        out_shape=jax.ShapeDtypeStruct((N, 4), jnp.float32),
    )(x, p0, p1, w1, b1, w2, b2, w3, b3)


def _cat_w1(W1):
    z = jnp.zeros((1, W1.shape[1]), W1.dtype)
    return jnp.concatenate([W1[0:3], z, W1[3:6], z, W1[6:10]], axis=0)


def kernel(x, edge_index, edge_attr,
           R1_W1, R1_b1, R1_W2, R1_b2, R1_W3, R1_b3,
           O_W1, O_b1, O_W2, O_b2, O_W3, O_b3,
           R2_W1, R2_b1, R2_W2, R2_b2, R2_W3, R2_b3):
    N = x.shape[0]
    E = edge_index.shape[1]

    xpad = jnp.concatenate([x, jnp.zeros((N, 1), x.dtype)], axis=1)
    dst2d = edge_index[1].reshape(E // _IW, _IW)
    src2d = edge_index[0].reshape(E // _IW, _IW)
    n_pad = ((N + _NS * 8 - 1) // (_NS * 8)) * (_NS * 8)
    zeros_n4 = jnp.zeros((n_pad, 4), jnp.float32)

    gi, gj = _sc_gather_pair(xpad, dst2d, src2d)
    emsg = _edge_mlp(
        gi, gj, edge_attr,
        _cat_w1(R1_W1), R1_b1.reshape(1, -1),
        R1_W2, R1_b2.reshape(1, -1),
        R1_W3, R1_b3.reshape(1, -1),
        sigmoid=False)

    parts = _sc_segment_sum(dst2d, emsg, zeros_n4)
    w3p = jnp.concatenate([O_W3, jnp.zeros((O_W3.shape[0], 1), O_W3.dtype)],
                          axis=1)
    b3p = jnp.concatenate([O_b3, jnp.zeros((1,), O_b3.dtype)])
    xt_pad = _node_mlp(
        x, parts[0, :N], parts[1, :N],
        O_W1, O_b1.reshape(1, -1),
        O_W2, O_b2.reshape(1, -1),
        w3p, b3p.reshape(1, -1))

    gi2, gj2 = _sc_gather_pair(xt_pad, dst2d, src2d)
    return _edge_mlp(
        gi2, gj2, emsg,
        _cat_w1(R2_W1), R2_b1.reshape(1, -1),
        R2_W2, R2_b2.reshape(1, -1),
        R2_W3, R2_b3.reshape(1, -1),
        sigmoid=True)
